# baseline (device time: 17411 ns/iter reference)
import jax
import jax.numpy as jnp
from jax import lax
from jax.experimental import pallas as pl
from jax.experimental.pallas import tpu as pltpu

N_DEV = 32
EPS = 1e-5


def kernel(x, gamma, beta):
    m, n = x.shape
    n_global = n * N_DEV

    def body(x_any, g_ref, b_ref, o_ref, x_ref, comm_ref, x_sem, send_sems, recv_sems):
        my = lax.axis_index("i")

        xcopy = pltpu.make_async_copy(x_any, x_ref, x_sem)
        xcopy.start()

        barrier_sem = pltpu.get_barrier_semaphore()
        for d in range(1, N_DEV):
            pl.semaphore_signal(
                barrier_sem, inc=1,
                device_id=(my ^ d,), device_id_type=pl.DeviceIdType.MESH,
            )

        xcopy.wait()
        xv = x_ref[:, :]
        comm_ref[0, 0, :] = jnp.sum(xv, axis=1)
        comm_ref[0, 1, :] = jnp.sum(xv * xv, axis=1)

        pl.semaphore_wait(barrier_sem, N_DEV - 1)

        rdmas = []
        for d in range(1, N_DEV):
            rdma = pltpu.make_async_remote_copy(
                src_ref=comm_ref.at[0],
                dst_ref=comm_ref.at[d],
                send_sem=send_sems.at[d],
                recv_sem=recv_sems.at[d],
                device_id=(my ^ d,),
                device_id_type=pl.DeviceIdType.MESH,
            )
            rdma.start()
            rdmas.append(rdma)
        for rdma in rdmas:
            rdma.wait()

        total = jnp.sum(comm_ref[:, :, :], axis=0)
        stats = jnp.transpose(total)
        mean = stats[:, 0:1] * (1.0 / n_global)
        var = stats[:, 1:2] * (1.0 / n_global) - mean * mean
        inv = lax.rsqrt(var + EPS)
        g2 = jnp.reshape(g_ref[:], (1, n))
        b2 = jnp.reshape(b_ref[:], (1, n))
        o_ref[:, :] = g2 * ((xv - mean) * inv) + b2

    return pl.pallas_call(
        body,
        out_shape=jax.ShapeDtypeStruct((m, n), x.dtype),
        in_specs=[
            pl.BlockSpec(memory_space=pl.ANY),
            pl.BlockSpec(memory_space=pltpu.VMEM),
            pl.BlockSpec(memory_space=pltpu.VMEM),
        ],
        out_specs=pl.BlockSpec(memory_space=pltpu.VMEM),
        scratch_shapes=[
            pltpu.VMEM((m, n), jnp.float32),
            pltpu.VMEM((N_DEV, 2, m), jnp.float32),
            pltpu.SemaphoreType.DMA,
            pltpu.SemaphoreType.DMA((N_DEV,)),
            pltpu.SemaphoreType.DMA((N_DEV,)),
        ],
        compiler_params=pltpu.CompilerParams(collective_id=0),
    )(x, gamma, beta)


# device time: 16928 ns/iter; 1.0285x vs baseline; 1.0285x over previous
import jax
import jax.numpy as jnp
from jax import lax
from jax.experimental import pallas as pl
from jax.experimental.pallas import tpu as pltpu

N_DEV = 32
EPS = 1e-5
_Z_OFFS = (8, 16, 24)
_PLANE_OFFS = tuple(range(1, 8))


def kernel(x, gamma, beta):
    m, n = x.shape
    n_global = n * N_DEV

    def body(x_ref, g_ref, b_ref, o_ref, comm1_ref, comm2_ref,
             send1_sems, recv1_sems, send2_sems, recv2_sems):
        my = lax.axis_index("i")

        barrier_sem = pltpu.get_barrier_semaphore()
        for off in _Z_OFFS + _PLANE_OFFS:
            pl.semaphore_signal(
                barrier_sem, inc=1,
                device_id=(my ^ off,), device_id_type=pl.DeviceIdType.MESH,
            )

        xv = x_ref[:, :]
        comm1_ref[0, 0, :] = jnp.sum(xv, axis=1)
        comm1_ref[0, 1, :] = jnp.sum(xv * xv, axis=1)

        pl.semaphore_wait(barrier_sem, len(_Z_OFFS) + len(_PLANE_OFFS))

        rdmas = []
        for f, off in enumerate(_Z_OFFS, start=1):
            rdma = pltpu.make_async_remote_copy(
                src_ref=comm1_ref.at[0],
                dst_ref=comm1_ref.at[f],
                send_sem=send1_sems.at[f],
                recv_sem=recv1_sems.at[f],
                device_id=(my ^ off,),
                device_id_type=pl.DeviceIdType.MESH,
            )
            rdma.start()
            rdmas.append(rdma)
        for rdma in rdmas:
            rdma.wait()
        comm2_ref[0, :, :] = jnp.sum(comm1_ref[:, :, :], axis=0)

        rdmas = []
        for e in _PLANE_OFFS:
            rdma = pltpu.make_async_remote_copy(
                src_ref=comm2_ref.at[0],
                dst_ref=comm2_ref.at[e],
                send_sem=send2_sems.at[e],
                recv_sem=recv2_sems.at[e],
                device_id=(my ^ e,),
                device_id_type=pl.DeviceIdType.MESH,
            )
            rdma.start()
            rdmas.append(rdma)
        for rdma in rdmas:
            rdma.wait()

        total = jnp.sum(comm2_ref[:, :, :], axis=0)
        stats = jnp.transpose(total)
        mean = stats[:, 0:1] * (1.0 / n_global)
        var = stats[:, 1:2] * (1.0 / n_global) - mean * mean
        inv = lax.rsqrt(var + EPS)
        g2 = jnp.reshape(g_ref[:], (1, n))
        b2 = jnp.reshape(b_ref[:], (1, n))
        o_ref[:, :] = g2 * ((xv - mean) * inv) + b2

    return pl.pallas_call(
        body,
        out_shape=jax.ShapeDtypeStruct((m, n), x.dtype),
        in_specs=[
            pl.BlockSpec(memory_space=pltpu.VMEM),
            pl.BlockSpec(memory_space=pltpu.VMEM),
            pl.BlockSpec(memory_space=pltpu.VMEM),
        ],
        out_specs=pl.BlockSpec(memory_space=pltpu.VMEM),
        scratch_shapes=[
            pltpu.VMEM((4, 2, m), jnp.float32),
            pltpu.VMEM((8, 2, m), jnp.float32),
            pltpu.SemaphoreType.DMA((4,)),
            pltpu.SemaphoreType.DMA((4,)),
            pltpu.SemaphoreType.DMA((8,)),
            pltpu.SemaphoreType.DMA((8,)),
        ],
        compiler_params=pltpu.CompilerParams(collective_id=0),
    )(x, gamma, beta)
